# baseline (speedup 1.0000x reference)
import functools
import jax
import jax.numpy as jnp
from jax import lax
from jax.experimental import pallas as pl
from jax.experimental.pallas import tpu as pltpu, tpu_sc as plsc

NC, NS = 2, 16
NW = NC * NS
R = 32
LANES = 16


def _sc_body(x_hbm, pe_hbm, pos_hbm, out_hbm, idx_v, bufx, bufp, sem):
    wid = lax.axis_index("s") * NC + lax.axis_index("c")
    rows_total = x_hbm.shape[0]
    D = x_hbm.shape[1]
    rows_per_w = rows_total // NW
    base = wid * rows_per_w

    def chunk(t, _):
        r0 = base + t * R
        pltpu.sync_copy(x_hbm.at[pl.ds(r0, R)], bufx)
        pltpu.sync_copy(pos_hbm.at[pl.ds(r0, R)], idx_v)
        pltpu.async_copy(pe_hbm.at[idx_v], bufp, sem).wait()

        def row(r, _):
            @plsc.parallel_loop(0, D // LANES, unroll=8)
            def col(j):
                c = j * LANES
                plsc.addupdate(bufx.at[r, pl.ds(c, LANES)],
                               bufp[r, pl.ds(c, LANES)])
            return ()

        lax.fori_loop(0, R, row, ())
        pltpu.sync_copy(bufx, out_hbm.at[pl.ds(r0, R)])
        return ()

    lax.fori_loop(0, rows_per_w // R, chunk, ())


def kernel(x, pe):
    B, L, D = x.shape
    x2 = x.reshape(B * L, D)
    pos = jnp.tile(jnp.arange(L, dtype=jnp.int32), B)
    run = pl.kernel(
        _sc_body,
        out_type=jax.ShapeDtypeStruct((B * L, D), x.dtype),
        mesh=plsc.VectorSubcoreMesh(core_axis_name="c", subcore_axis_name="s"),
        scratch_types=[
            pltpu.VMEM((R,), jnp.int32),
            pltpu.VMEM((R, D), jnp.float32),
            pltpu.VMEM((R, D), jnp.float32),
            pltpu.SemaphoreType.DMA,
        ],
    )
    return run(x2, pe, pos).reshape(B, L, D)


# speedup vs baseline: 2.2419x; 2.2419x over previous
import jax
import jax.numpy as jnp
from jax import lax
from jax.experimental import pallas as pl
from jax.experimental.pallas import tpu as pltpu, tpu_sc as plsc

NC, NS = 2, 16
NW = NC * NS
LS = 7168
L_BLK = 1024
R = 32
LANES = 16


def _tc_body(x_ref, pe_ref, o_ref):
    o_ref[0] = x_ref[0] + pe_ref[...]


def _sc_body(x_hbm, pe_hbm, out_hbm, bufx, bufp):
    B = 4
    L = pe_hbm.shape[0]
    D = pe_hbm.shape[1]
    rows_sc = L - LS
    per_w = (B * rows_sc) // NW
    wpb = NW // B

    wid = lax.axis_index("s") * NC + lax.axis_index("c")
    b = wid // wpb
    j = wid % wpb
    src0 = b * L + LS + j * per_w
    pe0 = LS + j * per_w
    out0 = wid * per_w

    def chunk(t, _):
        r0 = t * R
        pltpu.sync_copy(x_hbm.at[pl.ds(src0 + r0, R)], bufx)
        pltpu.sync_copy(pe_hbm.at[pl.ds(pe0 + r0, R)], bufp)

        def row(r, _):
            @plsc.parallel_loop(0, D // LANES, unroll=8)
            def col(k):
                c = k * LANES
                plsc.addupdate(bufx.at[r, pl.ds(c, LANES)],
                               bufp[r, pl.ds(c, LANES)])
            return ()

        lax.fori_loop(0, R, row, ())
        pltpu.sync_copy(bufx, out_hbm.at[pl.ds(out0 + r0, R)])
        return ()

    lax.fori_loop(0, per_w // R, chunk, ())


def kernel(x, pe):
    B, L, D = x.shape

    tc_out = pl.pallas_call(
        _tc_body,
        grid=(LS // L_BLK, B),
        in_specs=[
            pl.BlockSpec((1, L_BLK, D), lambda i, b: (b, i, 0)),
            pl.BlockSpec((L_BLK, D), lambda i, b: (i, 0)),
        ],
        out_specs=pl.BlockSpec((1, L_BLK, D), lambda i, b: (b, i, 0)),
        out_shape=jax.ShapeDtypeStruct((B, L, D), x.dtype),
    )(x, pe)

    rows_sc = L - LS
    sc_run = pl.kernel(
        _sc_body,
        out_type=jax.ShapeDtypeStruct((B * rows_sc, D), x.dtype),
        mesh=plsc.VectorSubcoreMesh(core_axis_name="c", subcore_axis_name="s"),
        scratch_types=[
            pltpu.VMEM((R, D), jnp.float32),
            pltpu.VMEM((R, D), jnp.float32),
        ],
    )
    sc_out = sc_run(x.reshape(B * L, D), pe)

    return lax.dynamic_update_slice(
        tc_out, sc_out.reshape(B, rows_sc, D), (0, LS, 0)
    )


# speedup vs baseline: 3.1532x; 1.4065x over previous
import jax
import jax.numpy as jnp
from jax.experimental import pallas as pl


L_BLK = 2048


def _body(x_ref, pe_ref, o_ref):
    o_ref[0] = x_ref[0] + pe_ref[...]


def kernel(x, pe):
    B, L, D = x.shape
    n_l = L // L_BLK
    return pl.pallas_call(
        _body,
        grid=(n_l, B),
        in_specs=[
            pl.BlockSpec((1, L_BLK, D), lambda i, b: (b, i, 0)),
            pl.BlockSpec((L_BLK, D), lambda i, b: (i, 0)),
        ],
        out_specs=pl.BlockSpec((1, L_BLK, D), lambda i, b: (b, i, 0)),
        out_shape=jax.ShapeDtypeStruct((B, L, D), x.dtype),
    )(x, pe[:L])
